# baseline (speedup 1.0000x reference)
import functools

import jax
import jax.numpy as jnp
from jax.experimental import pallas as pl
from jax.experimental.pallas import tpu as pltpu

_BS, _NN, _D = 2, 512, 128
_NUM_REL = 2

_CONTRACT0 = (((0,), (0,)), ((), ()))


def _split(v):
    vh = v.astype(jnp.bfloat16)
    vl = (v - vh.astype(jnp.float32)).astype(jnp.bfloat16)
    return vh, vl


def _mm3(xh, xl, wh, wl):
    return (jnp.dot(xh, wh, preferred_element_type=jnp.float32)
            + jnp.dot(xh, wl, preferred_element_type=jnp.float32)
            + jnp.dot(xl, wh, preferred_element_type=jnp.float32))


def _mm2(xh, xl, wh, wl):
    return (jnp.dot(xh, wh, preferred_element_type=jnp.float32)
            + jnp.dot(xh, wl, preferred_element_type=jnp.float32))


def _agg(a, yh):
    return jax.lax.dot_general(a, yh, _CONTRACT0,
                               preferred_element_type=jnp.float32)


def _rgcn_kernel(adj_hbm, x_ref, wrel0_ref, wroot0_ref, b0_ref,
                 wrel1_ref, wroot1_ref, b1_ref, out_ref, adj_scr, sems):
    copies = []
    for b in range(_BS):
        for t in range(2):
            c = pltpu.make_async_copy(adj_hbm.at[t, b], adj_scr.at[t, b],
                                      sems.at[2 * b + t])
            c.start()
            copies.append(c)

    ws = []
    for wrel_ref, wroot_ref, b_ref in ((wrel0_ref, wroot0_ref, b0_ref),
                                       (wrel1_ref, wroot1_ref, b1_ref)):
        ws.append((_split(wroot_ref[...]), _split(wrel_ref[0]),
                   _split(wrel_ref[1]), b_ref[...]))

    for b in range(_BS):
        copies[2 * b].wait()
        copies[2 * b + 1].wait()
        aug = adj_scr[0, b]
        pun = adj_scr[1, b]
        m1 = aug == 1
        m0 = (pun == 1) & (aug != 1)
        a1 = m1.astype(jnp.bfloat16)
        a0 = m0.astype(jnp.bfloat16)

        inv0 = 1.0 / jnp.maximum(jnp.sum(m0.astype(jnp.float32), axis=0), 1.0)
        inv1 = 1.0 / jnp.maximum(jnp.sum(m1.astype(jnp.float32), axis=0), 1.0)

        x = x_ref[b]
        for (wrh, wrl), (w0h, w0l), (w1h, w1l), bias in ws:
            xh, xl = _split(x)
            hroot = _mm3(xh, xl, wrh, wrl) + bias
            y0h = _mm2(xh, xl, w0h, w0l).astype(jnp.bfloat16)
            y1h = _mm2(xh, xl, w1h, w1l).astype(jnp.bfloat16)
            h = (hroot + _agg(a0, y0h) * inv0[:, None]
                 + _agg(a1, y1h) * inv1[:, None])
            x = jnp.where(h > 0, h, jnp.exp(jnp.minimum(h, 0.0)) - 1.0)
        out_ref[b] = x


@functools.partial(jax.jit, static_argnames=())
def _run(adj, x, wrel0, wroot0, b0, wrel1, wroot1, b1):
    vmem = functools.partial(pl.BlockSpec, memory_space=pltpu.VMEM)
    return pl.pallas_call(
        _rgcn_kernel,
        in_specs=[
            pl.BlockSpec(memory_space=pl.ANY),
            vmem(), vmem(), vmem(), vmem(), vmem(), vmem(), vmem(),
        ],
        out_specs=vmem(),
        out_shape=jax.ShapeDtypeStruct((_BS, _NN, _D), jnp.float32),
        scratch_shapes=[
            pltpu.VMEM((2, _BS, _NN, _NN), jnp.int32),
            pltpu.SemaphoreType.DMA((2 * _BS,)),
        ],
    )(adj, x, wrel0, wroot0, b0, wrel1, wroot1, b1)


def kernel(feature_list, adj_list, aug_pun_adj, pooled_output, p_nodes_mask,
           o_nodes_mask, W_rel0, W_root0, bias0, W_rel1, W_root1, bias1):
    x = feature_list[0]
    adj = aug_pun_adj.astype(jnp.int32)
    out = _run(adj, x, W_rel0, W_root0, bias0.reshape(1, _D),
               W_rel1, W_root1, bias1.reshape(1, _D))
    return out


# speedup vs baseline: 1.3102x; 1.3102x over previous
import functools

import jax
import jax.numpy as jnp
from jax.experimental import pallas as pl

_BS, _NN, _D = 2, 512, 128
_NUM_REL = 2

_CONTRACT0 = (((0,), (0,)), ((), ()))


def _split(v):
    vh = v.astype(jnp.bfloat16)
    vl = (v - vh.astype(jnp.float32)).astype(jnp.bfloat16)
    return vh, vl


def _mm3(xh, xl, wh, wl):
    return (jnp.dot(xh, wh, preferred_element_type=jnp.float32)
            + jnp.dot(xh, wl, preferred_element_type=jnp.float32)
            + jnp.dot(xl, wh, preferred_element_type=jnp.float32))


def _mm1(xh, wh):
    return jnp.dot(xh, wh, preferred_element_type=jnp.float32)


def _agg(a, yh):
    return jax.lax.dot_general(a, yh, _CONTRACT0,
                               preferred_element_type=jnp.float32)


def _rgcn_kernel(adj_ref, x_ref, wrel0_ref, wroot0_ref, b0_ref,
                 wrel1_ref, wroot1_ref, b1_ref, out_ref):
    aug = adj_ref[0, 0]
    pun = adj_ref[1, 0]
    m1 = aug == 1
    m0 = (pun == 1) & (aug != 1)
    a1 = m1.astype(jnp.bfloat16)
    a0 = m0.astype(jnp.bfloat16)

    inv0 = 1.0 / jnp.maximum(jnp.sum(m0.astype(jnp.float32), axis=0), 1.0)
    inv1 = 1.0 / jnp.maximum(jnp.sum(m1.astype(jnp.float32), axis=0), 1.0)

    x = x_ref[0]
    for wrel_ref, wroot_ref, b_ref in ((wrel0_ref, wroot0_ref, b0_ref),
                                       (wrel1_ref, wroot1_ref, b1_ref)):
        wrh, wrl = _split(wroot_ref[...])
        w0h = wrel_ref[0].astype(jnp.bfloat16)
        w1h = wrel_ref[1].astype(jnp.bfloat16)
        xh, xl = _split(x)
        hroot = _mm3(xh, xl, wrh, wrl) + b_ref[...]
        y0h = _mm1(xh, w0h).astype(jnp.bfloat16)
        y1h = _mm1(xh, w1h).astype(jnp.bfloat16)
        h = (hroot + _agg(a0, y0h) * inv0[:, None]
             + _agg(a1, y1h) * inv1[:, None])
        x = jnp.where(h > 0, h, jnp.exp(jnp.minimum(h, 0.0)) - 1.0)
    out_ref[0] = x


@functools.partial(jax.jit, static_argnames=())
def _run(adj, x, wrel0, wroot0, b0, wrel1, wroot1, b1):
    return pl.pallas_call(
        _rgcn_kernel,
        grid=(_BS,),
        in_specs=[
            pl.BlockSpec((2, 1, _NN, _NN), lambda b: (0, b, 0, 0)),
            pl.BlockSpec((1, _NN, _D), lambda b: (b, 0, 0)),
            pl.BlockSpec((_NUM_REL, _D, _D), lambda b: (0, 0, 0)),
            pl.BlockSpec((_D, _D), lambda b: (0, 0)),
            pl.BlockSpec((1, _D), lambda b: (0, 0)),
            pl.BlockSpec((_NUM_REL, _D, _D), lambda b: (0, 0, 0)),
            pl.BlockSpec((_D, _D), lambda b: (0, 0)),
            pl.BlockSpec((1, _D), lambda b: (0, 0)),
        ],
        out_specs=pl.BlockSpec((1, _NN, _D), lambda b: (b, 0, 0)),
        out_shape=jax.ShapeDtypeStruct((_BS, _NN, _D), jnp.float32),
    )(adj, x, wrel0, wroot0, b0, wrel1, wroot1, b1)


def kernel(feature_list, adj_list, aug_pun_adj, pooled_output, p_nodes_mask,
           o_nodes_mask, W_rel0, W_root0, bias0, W_rel1, W_root1, bias1):
    x = feature_list[0]
    adj = aug_pun_adj.astype(jnp.int32)
    out = _run(adj, x, W_rel0, W_root0, bias0.reshape(1, _D),
               W_rel1, W_root1, bias1.reshape(1, _D))
    return out
